# Initial kernel scaffold; baseline (speedup 1.0000x reference)
import jax
import jax.numpy as jnp
from jax.experimental import pallas as pl


def kernel(x, pos, edge_index, cell_offset, unit_cell, batch, W_emb, b_emb, msg_W1, msg_b1, msg_g1, msg_be1, msg_W2, msg_b2, msg_g2, msg_be2, upd_W1, upd_b1, upd_g1, upd_be1, upd_W2, upd_b2, upd_g2, upd_be2, pred_W1, pred_b1, pred_W2, pred_b2):
    raise NotImplementedError("write your pallas kernel here")


# speedup vs baseline: 2.1623x; 2.1623x over previous
import functools

import jax
import jax.numpy as jnp
from jax import lax
from jax.experimental import pallas as pl
from jax.experimental.pallas import tpu as pltpu
from jax.experimental.pallas import tpu_sc as plsc

N = 50000
E = 800000
H = 64
NC = 2
NS = 16
NW = NC * NS
CHUNK = 128
IDX_PAD = NW * CHUNK * 392
HALF = N // 2
ACC_ROWS = 25088
TRASH = HALF
PER_TILE_ROWS = ACC_ROWS // NS
EDGES_PER_TILE = E // NS
SCCH = 391
EDGE_TAIL = EDGES_PER_TILE - (SCCH - 1) * CHUNK
OUT_PER_TILE = 1563

def _mesh():
    return plsc.VectorSubcoreMesh(core_axis_name="c", subcore_axis_name="s")


def _sc_gather(table, idx_pad, d):
    per_w = IDX_PAD // NW
    nch = per_w // CHUNK

    @functools.partial(
        pl.kernel,
        mesh=_mesh(),
        compiler_params=pltpu.CompilerParams(use_tc_tiling_on_sc=False),
        out_type=jax.ShapeDtypeStruct((IDX_PAD, d), jnp.float32),
        scratch_types=[
            pltpu.VMEM((per_w,), jnp.int32),
            pltpu.VMEM((CHUNK, d), jnp.float32),
            pltpu.VMEM((CHUNK, d), jnp.float32),
            pltpu.SemaphoreType.DMA,
            pltpu.SemaphoreType.DMA,
        ],
    )
    def k(table_hbm, idx_hbm, out_hbm, idx_v, buf0, buf1, sem0, sem1):
        wid = lax.axis_index("s") * NC + lax.axis_index("c")
        base = wid * per_w
        pltpu.sync_copy(idx_hbm.at[pl.ds(base, per_w)], idx_v)

        @pl.loop(0, nch // 2)
        def _(jj):
            j = jj * 2
            c0 = pltpu.async_copy(
                table_hbm.at[idx_v.at[pl.ds(j * CHUNK, CHUNK)]], buf0, sem0)
            c1 = pltpu.async_copy(
                table_hbm.at[idx_v.at[pl.ds((j + 1) * CHUNK, CHUNK)]], buf1, sem1)
            c0.wait()
            pltpu.sync_copy(buf0, out_hbm.at[pl.ds(base + j * CHUNK, CHUNK)])
            c1.wait()
            pltpu.sync_copy(buf1, out_hbm.at[pl.ds(base + (j + 1) * CHUNK, CHUNK)])

    return k(table, idx_pad)


STAGE_CH = 32
STAGE_W = STAGE_CH * CHUNK
NSTAGE = 13
TAIL_W = EDGES_PER_TILE - 12 * STAGE_W
TAIL_CH = 7


def _sc_scatter_add(m, dst):
    @functools.partial(
        pl.kernel,
        mesh=_mesh(),
        compiler_params=pltpu.CompilerParams(use_tc_tiling_on_sc=False),
        out_type=jax.ShapeDtypeStruct((N, H), jnp.float32),
        scratch_types=[
            pltpu.VMEM((STAGE_W,), jnp.int32),
            pltpu.VMEM((STAGE_CH, CHUNK), jnp.int32),
            pltpu.VMEM((CHUNK, H), jnp.float32),
            pltpu.VMEM((CHUNK, H), jnp.float32),
            pltpu.VMEM_SHARED((ACC_ROWS, H), jnp.float32),
            pltpu.SemaphoreType.DMA,
            pltpu.SemaphoreType.DMA,
            pltpu.SemaphoreType.DMA,
            pltpu.SemaphoreType.DMA,
        ],
    )
    def k(m_hbm, dst_hbm, out_hbm, dstv, idx2, mb0, mb1, acc,
          semL0, semL1, semS0, semS1):
        c = lax.axis_index("c")
        s = lax.axis_index("s")
        core_base = c * HALF

        zv = jnp.zeros((16,), jnp.float32)

        @pl.loop(0, CHUNK)
        def _(r):
            @pl.loop(0, H // 16)
            def _(q):
                mb0[r, pl.ds(q * 16, 16)] = zv

        zbase = s * PER_TILE_ROWS
        for t in range(PER_TILE_ROWS // CHUNK):
            pltpu.sync_copy(mb0, acc.at[pl.ds(zbase + t * CHUNK, CHUNK)])
        rem = PER_TILE_ROWS % CHUNK
        if rem:
            pltpu.sync_copy(mb0.at[pl.ds(0, rem)],
                            acc.at[pl.ds(zbase + (PER_TILE_ROWS // CHUNK) * CHUNK, rem)])

        plsc.subcore_barrier()

        ebase = s * EDGES_PER_TILE
        mbufs = (mb0, mb1)
        semLs = (semL0, semL1)
        semSs = (semS0, semS1)

        def do_stage(t, nch, nids):
            sbase = ebase + t * STAGE_W
            pltpu.sync_copy(dst_hbm.at[pl.ds(sbase, nids)],
                            dstv.at[pl.ds(0, nids)])

            @pl.loop(0, STAGE_CH)
            def _(j):
                @pl.loop(0, CHUNK // 16)
                def _(q):
                    w = j * CHUNK + q * 16
                    ok_e = (w + lax.iota(jnp.int32, 16)) < nids
                    v = dstv[pl.ds(jnp.minimum(w, STAGE_W - 16), 16)]
                    lo = v - core_base
                    ok = (lo >= 0) & (lo < HALF) & ok_e
                    idx2[j, pl.ds(q * 16, 16)] = jnp.where(ok, lo, TRASH)

            def load(j):
                nrows = min(CHUNK, nids - j * CHUNK)
                return pltpu.async_copy(
                    m_hbm.at[pl.ds(sbase + j * CHUNK, nrows)],
                    mbufs[j % 2].at[pl.ds(0, nrows)], semLs[j % 2])

            pend = [None, None]
            loads = [None, None]
            for j in range(min(2, nch)):
                loads[j % 2] = load(j)
            for j in range(nch):
                p = j % 2
                loads[p].wait()
                pend[p] = pltpu.async_copy(mbufs[p], acc.at[idx2.at[j]],
                                           semSs[p], add=True)
                if j + 2 < nch:
                    pend[p].wait()
                    pend[p] = None
                    loads[p] = load(j + 2)
            for cs in pend:
                if cs is not None:
                    cs.wait()

        for t in range(NSTAGE - 1):
            do_stage(t, STAGE_CH, STAGE_W)
        do_stage(NSTAGE - 1, TAIL_CH, TAIL_W)

        plsc.subcore_barrier()

        start = jnp.minimum(s * OUT_PER_TILE, HALF - OUT_PER_TILE)
        pltpu.sync_copy(acc.at[pl.ds(start, OUT_PER_TILE)],
                        out_hbm.at[pl.ds(core_base + start, OUT_PER_TILE)])

    return k(m, dst)


def _ln(v, g, b):
    mu = jnp.mean(v, axis=-1, keepdims=True)
    var = jnp.mean((v - mu) ** 2, axis=-1, keepdims=True)
    return (v - mu) * jax.lax.rsqrt(var + 1e-5) * g + b


def _emb_kernel(x_ref, w_ref, b_ref, o_ref):
    o_ref[...] = jnp.dot(x_ref[...], w_ref[...].T,
                         preferred_element_type=jnp.float32) + b_ref[...]


def _dist_kernel(ps_ref, pd_ref, co_ref, uc_ref, o_ref):
    pbc = jnp.dot(co_ref[...], uc_ref[...], preferred_element_type=jnp.float32)
    dvec = (pd_ref[...][:, :3] - pbc) - ps_ref[...][:, :3]
    o_ref[...] = jnp.sqrt(jnp.sum(dvec * dvec, axis=-1, keepdims=True))


def _edge_mlp_kernel(hs_ref, hd_ref, dist_ref, w1a_ref, w1b_ref, w1c_ref,
                     b1_ref, g1_ref, be1_ref, w2_ref, b2_ref, g2_ref, be2_ref,
                     o_ref):
    m = (jnp.dot(hd_ref[...], w1a_ref[...].T, preferred_element_type=jnp.float32)
         + jnp.dot(hs_ref[...], w1b_ref[...].T, preferred_element_type=jnp.float32)
         + dist_ref[...] * w1c_ref[...] + b1_ref[...])
    m = jax.nn.relu(_ln(m, g1_ref[...], be1_ref[...]))
    m = jnp.dot(m, w2_ref[...].T, preferred_element_type=jnp.float32) + b2_ref[...]
    o_ref[...] = jax.nn.relu(_ln(m, g2_ref[...], be2_ref[...]))


def _node_mlp_kernel(h_ref, a_ref, w1a_ref, w1b_ref, b1_ref, g1_ref, be1_ref,
                     w2_ref, b2_ref, g2_ref, be2_ref, o_ref):
    u = (jnp.dot(h_ref[...], w1a_ref[...].T, preferred_element_type=jnp.float32)
         + jnp.dot(a_ref[...], w1b_ref[...].T, preferred_element_type=jnp.float32)
         + b1_ref[...])
    u = jax.nn.relu(_ln(u, g1_ref[...], be1_ref[...]))
    u = jnp.dot(u, w2_ref[...].T, preferred_element_type=jnp.float32) + b2_ref[...]
    u = jax.nn.relu(_ln(u, g2_ref[...], be2_ref[...]))
    o_ref[...] = h_ref[...] + u


def _pool_head_kernel(h_ref, b3_ref, pw1_ref, pb1_ref, pw2_ref, pb2_ref,
                      o_ref, acc_ref):
    i = pl.program_id(0)

    @pl.when(i == 0)
    def _():
        acc_ref[...] = jnp.zeros_like(acc_ref)

    bv = b3_ref[0, 0, :]
    gids = lax.broadcasted_iota(jnp.int32, (16, bv.shape[0]), 0)
    oh = (gids == bv[None, :]).astype(jnp.float32)
    acc_ref[...] += jnp.dot(oh, h_ref[...], preferred_element_type=jnp.float32)

    @pl.when(i == pl.num_programs(0) - 1)
    def _():
        pooled = acc_ref[...]
        p1 = jax.nn.relu(jnp.dot(pooled, pw1_ref[...].T,
                                 preferred_element_type=jnp.float32) + pb1_ref[...])
        o_ref[...] = (jnp.sum(p1 * pw2_ref[...], axis=-1, keepdims=True)
                      + pb2_ref[...])


BN = 5000
BE = 4000
NBLK = N // BN
EBLK = E // BE


def _row(v):
    return v.reshape(1, -1)


def _full(a):
    return pl.BlockSpec(a.shape, lambda *_: tuple(0 for _ in a.shape))


def kernel(x, pos, edge_index, cell_offset, unit_cell, batch, W_emb, b_emb,
           msg_W1, msg_b1, msg_g1, msg_be1, msg_W2, msg_b2, msg_g2, msg_be2,
           upd_W1, upd_b1, upd_g1, upd_be1, upd_W2, upd_b2, upd_g2, upd_be2,
           pred_W1, pred_b1, pred_W2, pred_b2):
    f32 = jnp.float32
    idx_flat = edge_index.reshape(2 * E)
    idx_pad = jnp.concatenate(
        [idx_flat, jnp.zeros((IDX_PAD - 2 * E,), jnp.int32)])
    dst = edge_index[1]

    h = pl.pallas_call(
        _emb_kernel,
        grid=(NBLK,),
        in_specs=[pl.BlockSpec((BN, x.shape[1]), lambda i: (i, 0)),
                  _full(W_emb), _full(_row(b_emb))],
        out_specs=pl.BlockSpec((BN, H), lambda i: (i, 0)),
        out_shape=jax.ShapeDtypeStruct((N, H), f32),
    )(x, W_emb, _row(b_emb))

    pos_pad = jnp.pad(pos, ((0, 0), (0, 13)))
    pg = _sc_gather(pos_pad, idx_pad, 16)
    dist = pl.pallas_call(
        _dist_kernel,
        grid=(EBLK,),
        in_specs=[pl.BlockSpec((BE, 16), lambda i: (i, 0)),
                  pl.BlockSpec((BE, 16), lambda i: (EBLK + i, 0)),
                  pl.BlockSpec((BE, 3), lambda i: (i, 0)),
                  _full(unit_cell)],
        out_specs=pl.BlockSpec((BE, 1), lambda i: (i, 0)),
        out_shape=jax.ShapeDtypeStruct((E, 1), f32),
    )(pg, pg, cell_offset, unit_cell)

    n_layers = msg_W1.shape[0]
    wspec = pl.BlockSpec((H, H), lambda i: (0, 0))
    rspec = pl.BlockSpec((1, H), lambda i: (0, 0))
    for l in range(n_layers):
        hg = _sc_gather(h, idx_pad, H)
        w1 = msg_W1[l]
        m = pl.pallas_call(
            _edge_mlp_kernel,
            grid=(EBLK,),
            in_specs=[pl.BlockSpec((BE, H), lambda i: (i, 0)),
                      pl.BlockSpec((BE, H), lambda i: (EBLK + i, 0)),
                      pl.BlockSpec((BE, 1), lambda i: (i, 0)),
                      wspec, wspec, rspec, rspec, rspec, rspec,
                      wspec, rspec, rspec, rspec],
            out_specs=pl.BlockSpec((BE, H), lambda i: (i, 0)),
            out_shape=jax.ShapeDtypeStruct((E, H), f32),
        )(hg, hg, dist, w1[:, :H], w1[:, H:2 * H], _row(w1[:, 2 * H]),
          _row(msg_b1[l]), _row(msg_g1[l]), _row(msg_be1[l]),
          msg_W2[l], _row(msg_b2[l]), _row(msg_g2[l]), _row(msg_be2[l]))

        aggr = _sc_scatter_add(m, dst)

        u1 = upd_W1[l]
        h = pl.pallas_call(
            _node_mlp_kernel,
            grid=(NBLK,),
            in_specs=[pl.BlockSpec((BN, H), lambda i: (i, 0)),
                      pl.BlockSpec((BN, H), lambda i: (i, 0)),
                      wspec, wspec, rspec, rspec, rspec,
                      wspec, rspec, rspec, rspec],
            out_specs=pl.BlockSpec((BN, H), lambda i: (i, 0)),
            out_shape=jax.ShapeDtypeStruct((N, H), f32),
        )(h, aggr, u1[:, :H], u1[:, H:],
          _row(upd_b1[l]), _row(upd_g1[l]), _row(upd_be1[l]),
          upd_W2[l], _row(upd_b2[l]), _row(upd_g2[l]), _row(upd_be2[l]))

    batch3 = batch.reshape(NBLK, 1, BN)
    out = pl.pallas_call(
        _pool_head_kernel,
        grid=(NBLK,),
        in_specs=[pl.BlockSpec((BN, H), lambda i: (i, 0)),
                  pl.BlockSpec((1, 1, BN), lambda i: (i, 0, 0)),
                  pl.BlockSpec((H, H), lambda i: (0, 0)),
                  pl.BlockSpec((1, H), lambda i: (0, 0)),
                  pl.BlockSpec((1, H), lambda i: (0, 0)),
                  pl.BlockSpec((1, 1), lambda i: (0, 0))],
        out_specs=pl.BlockSpec((16, 1), lambda i: (0, 0)),
        out_shape=jax.ShapeDtypeStruct((16, 1), f32),
        scratch_shapes=[pltpu.VMEM((16, H), f32)],
    )(h, batch3, pred_W1, _row(pred_b1), pred_W2, _row(pred_b2))
    return out
